# baseline (speedup 1.0000x reference)
import functools

import jax
import jax.numpy as jnp
from jax import lax
from jax.experimental import pallas as pl
from jax.experimental.pallas import tpu as pltpu
from jax.experimental.pallas import tpu_sc as plsc

USER_DIM = 1000000
EMBED_DIM = 64
OUT_DIM = 64
BATCH = 16384

NC, NS = 2, 16
NW = NC * NS
B_PER_W = BATCH // NW
CHUNK = 128
N_CHUNKS = B_PER_W // CHUNK

_mesh = plsc.VectorSubcoreMesh(
    core_axis_name="c", subcore_axis_name="s", num_cores=NC, num_subcores=NS
)


@functools.partial(
    pl.kernel,
    out_type=jax.ShapeDtypeStruct((BATCH, EMBED_DIM), jnp.float32),
    mesh=_mesh,
    scratch_types=[
        pltpu.VMEM((N_CHUNKS, CHUNK), jnp.int32),
        pltpu.VMEM((B_PER_W, EMBED_DIM), jnp.float32),
        pltpu.SemaphoreType.DMA,
    ],
    compiler_params=pltpu.CompilerParams(use_tc_tiling_on_sc=False),
)
def _sc_gather(idx_hbm, table_hbm, emb_hbm, idx_v, rows_v, sem):
    wid = lax.axis_index("s") * NC + lax.axis_index("c")
    base = wid * B_PER_W
    pltpu.sync_copy(idx_hbm.at[wid], idx_v)
    copies = [
        pltpu.async_copy(
            table_hbm.at[idx_v.at[j]],
            rows_v.at[pl.ds(j * CHUNK, CHUNK)],
            sem,
        )
        for j in range(N_CHUNKS)
    ]
    for c in copies:
        c.wait()
    pltpu.sync_copy(rows_v, emb_hbm.at[pl.ds(base, B_PER_W)])


BM = 2048


def _mm_body(emb_ref, w_ref, b_ref, out_ref):
    out_ref[...] = (
        jnp.dot(emb_ref[...], w_ref[...], preferred_element_type=jnp.float32)
        + b_ref[...]
    )


def _tc_linear(emb, W, b2d):
    return pl.pallas_call(
        _mm_body,
        grid=(BATCH // BM,),
        in_specs=[
            pl.BlockSpec((BM, EMBED_DIM), lambda i: (i, 0)),
            pl.BlockSpec((EMBED_DIM, OUT_DIM), lambda i: (0, 0)),
            pl.BlockSpec((1, OUT_DIM), lambda i: (0, 0)),
        ],
        out_specs=pl.BlockSpec((BM, OUT_DIM), lambda i: (i, 0)),
        out_shape=jax.ShapeDtypeStruct((BATCH, OUT_DIM), jnp.float32),
    )(emb, W, b2d)


def kernel(x, table, W, b):
    idx = x.astype(jnp.int32).reshape(NW, N_CHUNKS, CHUNK)
    emb = _sc_gather(idx, table)
    return _tc_linear(emb, W, b.reshape(1, OUT_DIM))


# speedup vs baseline: 1.0009x; 1.0009x over previous
import functools

import jax
import jax.numpy as jnp
from jax import lax
from jax.experimental import pallas as pl
from jax.experimental.pallas import tpu as pltpu
from jax.experimental.pallas import tpu_sc as plsc

USER_DIM = 1000000
EMBED_DIM = 64
OUT_DIM = 64
BATCH = 16384

NC, NS = 2, 16
NW = NC * NS
B_PER_W = BATCH // NW
CHUNK = 128
N_CHUNKS = B_PER_W // CHUNK

_mesh = plsc.VectorSubcoreMesh(
    core_axis_name="c", subcore_axis_name="s", num_cores=NC, num_subcores=NS
)


@functools.partial(
    pl.kernel,
    out_type=jax.ShapeDtypeStruct((BATCH, 128), jnp.float32),
    mesh=_mesh,
    scratch_types=[
        pltpu.VMEM((N_CHUNKS, CHUNK), jnp.int32),
        pltpu.VMEM((B_PER_W, 128), jnp.float32),
        pltpu.SemaphoreType.DMA,
    ],
)
def _sc_gather(idx_hbm, table2_hbm, emb_hbm, idx_v, rows_v, sem):
    wid = lax.axis_index("s") * NC + lax.axis_index("c")
    base = wid * B_PER_W
    pltpu.sync_copy(idx_hbm.at[wid], idx_v)
    copies = [
        pltpu.async_copy(
            table2_hbm.at[idx_v.at[j]],
            rows_v.at[pl.ds(j * CHUNK, CHUNK)],
            sem,
        )
        for j in range(N_CHUNKS)
    ]
    for c in copies:
        c.wait()
    pltpu.sync_copy(rows_v, emb_hbm.at[pl.ds(base, B_PER_W)])


BM = 2048


def _mm_body(emb_ref, par_ref, ww_ref, b_ref, out_ref):
    lanes = lax.broadcasted_iota(jnp.int32, (BM, 128), 1)
    par = par_ref[...]
    sel = jnp.where(lanes < EMBED_DIM, 1.0 - par, par)
    out_ref[...] = (
        jnp.dot(emb_ref[...] * sel, ww_ref[...], preferred_element_type=jnp.float32)
        + b_ref[...]
    )


def _tc_linear(emb, par, WW, b2d):
    return pl.pallas_call(
        _mm_body,
        grid=(BATCH // BM,),
        in_specs=[
            pl.BlockSpec((BM, 128), lambda i: (i, 0)),
            pl.BlockSpec((BM, 1), lambda i: (i, 0)),
            pl.BlockSpec((128, OUT_DIM), lambda i: (0, 0)),
            pl.BlockSpec((1, OUT_DIM), lambda i: (0, 0)),
        ],
        out_specs=pl.BlockSpec((BM, OUT_DIM), lambda i: (i, 0)),
        out_shape=jax.ShapeDtypeStruct((BATCH, OUT_DIM), jnp.float32),
    )(emb, par, WW, b2d)


def kernel(x, table, W, b):
    xi = x.astype(jnp.int32)
    idx2 = (xi >> 1).reshape(NW, N_CHUNKS, CHUNK)
    table2 = table.reshape(USER_DIM // 2, 2 * EMBED_DIM)
    emb = _sc_gather(idx2, table2)
    par = (xi & 1).astype(jnp.float32).reshape(BATCH, 1)
    WW = jnp.concatenate([W, W], axis=0)
    return _tc_linear(emb, par, WW, b.reshape(1, OUT_DIM))


# speedup vs baseline: 1.3052x; 1.3040x over previous
import functools

import jax
import jax.numpy as jnp
from jax import lax
from jax.experimental import pallas as pl
from jax.experimental.pallas import tpu as pltpu
from jax.experimental.pallas import tpu_sc as plsc

USER_DIM = 1000000
EMBED_DIM = 64
OUT_DIM = 64
BATCH = 16384

NC, NS = 2, 16
NW = NC * NS
B_PER_W = BATCH // NW
CHUNK = 128
N_CHUNKS = B_PER_W // CHUNK

_mesh = plsc.VectorSubcoreMesh(
    core_axis_name="c", subcore_axis_name="s", num_cores=NC, num_subcores=NS
)


TBK = 2048


N_TBLK = (USER_DIM + TBK - 1) // TBK
T2_ROWS = N_TBLK * (TBK // 2)


def _tr_body(tT_ref, out_ref):
    t = jnp.swapaxes(tT_ref[...], 0, 1)
    out_ref[:, 0:EMBED_DIM] = t[0 : TBK // 2, :]
    out_ref[:, EMBED_DIM:128] = t[TBK // 2 : TBK, :]


def _tc_transpose(tableT):
    return pl.pallas_call(
        _tr_body,
        grid=(N_TBLK,),
        in_specs=[pl.BlockSpec((EMBED_DIM, TBK), lambda i: (0, i))],
        out_specs=pl.BlockSpec((TBK // 2, 128), lambda i: (i, 0)),
        out_shape=jax.ShapeDtypeStruct((T2_ROWS, 128), jnp.float32),
    )(tableT)


@functools.partial(
    pl.kernel,
    out_type=jax.ShapeDtypeStruct((BATCH, 128), jnp.float32),
    mesh=_mesh,
    scratch_types=[
        pltpu.VMEM((N_CHUNKS, CHUNK), jnp.int32),
        pltpu.VMEM((B_PER_W, 128), jnp.float32),
        pltpu.SemaphoreType.DMA,
    ],
)
def _sc_gather(idx_hbm, table2_hbm, emb_hbm, idx_v, rows_v, sem):
    wid = lax.axis_index("s") * NC + lax.axis_index("c")
    base = wid * B_PER_W
    pltpu.sync_copy(idx_hbm.at[wid], idx_v)
    copies = [
        pltpu.async_copy(
            table2_hbm.at[idx_v.at[j]],
            rows_v.at[pl.ds(j * CHUNK, CHUNK)],
            sem,
        )
        for j in range(N_CHUNKS)
    ]
    for c in copies:
        c.wait()
    pltpu.sync_copy(rows_v, emb_hbm.at[pl.ds(base, B_PER_W)])


BM = 2048


def _mm_body(emb_ref, par_ref, ww_ref, b_ref, out_ref):
    lanes = lax.broadcasted_iota(jnp.int32, (BM, 128), 1)
    par = par_ref[...]
    sel = jnp.where(lanes < EMBED_DIM, 1.0 - par, par)
    out_ref[...] = (
        jnp.dot(emb_ref[...] * sel, ww_ref[...], preferred_element_type=jnp.float32)
        + b_ref[...]
    )


def _tc_linear(emb, par, WW, b2d):
    return pl.pallas_call(
        _mm_body,
        grid=(BATCH // BM,),
        in_specs=[
            pl.BlockSpec((BM, 128), lambda i: (i, 0)),
            pl.BlockSpec((BM, 1), lambda i: (i, 0)),
            pl.BlockSpec((128, OUT_DIM), lambda i: (0, 0)),
            pl.BlockSpec((1, OUT_DIM), lambda i: (0, 0)),
        ],
        out_specs=pl.BlockSpec((BM, OUT_DIM), lambda i: (i, 0)),
        out_shape=jax.ShapeDtypeStruct((BATCH, OUT_DIM), jnp.float32),
    )(emb, par, WW, b2d)


def kernel(x, table, W, b):
    xi = x.astype(jnp.int32)
    g = ((xi >> 11) << 10) | (xi & 1023)
    idx2 = g.reshape(NW, N_CHUNKS, CHUNK)
    table2 = _tc_transpose(table.T)
    emb = _sc_gather(idx2, table2)
    par = ((xi >> 10) & 1).astype(jnp.float32).reshape(BATCH, 1)
    WW = jnp.concatenate([W, W], axis=0)
    return _tc_linear(emb, par, WW, b.reshape(1, OUT_DIM))
